# Initial kernel scaffold; baseline (speedup 1.0000x reference)
import jax
import jax.numpy as jnp
from jax.experimental import pallas as pl


def kernel(x, local_num_frames, W1, b1, W2, b2):
    raise NotImplementedError("write your pallas kernel here")


# speedup vs baseline: 2.7746x; 2.7746x over previous
import functools
import numpy as np
import jax
import jax.numpy as jnp
from jax import lax
from jax.experimental import pallas as pl
from jax.experimental.pallas import tpu as pltpu

_MM_HIDDEN = 1024
_HW = 24
_HEADS = 16
_T_MAX = 16
_HDIM = _MM_HIDDEN // _HEADS

_INTERPRET = False


def _pos_1d(embed_dim, pos):
    omega = np.arange(embed_dim // 2, dtype=np.float32)
    omega /= embed_dim / 2.0
    omega = 1.0 / 10000 ** omega
    pos = pos.reshape(-1)
    out = np.einsum('m,d->md', pos, omega)
    return np.concatenate([np.sin(out), np.cos(out)], axis=1)


def _pos_2d(embed_dim, grid):
    emb_h = _pos_1d(embed_dim // 2, grid[0])
    emb_w = _pos_1d(embed_dim // 2, grid[1])
    return np.concatenate([emb_h, emb_w], axis=1)


def _pos_3d(embed_dim, grid_size, t_size):
    embed_dim_spatial = embed_dim // 4 * 3
    embed_dim_temporal = embed_dim // 4
    grid_h = np.arange(grid_size, dtype=np.float32)
    grid_w = np.arange(grid_size, dtype=np.float32)
    grid = np.meshgrid(grid_w, grid_h)
    grid = np.stack(grid, axis=0).reshape([2, 1, grid_size, grid_size])
    pe_s = _pos_2d(embed_dim_spatial, grid)
    grid_t = np.arange(t_size, dtype=np.float32)
    pe_t = _pos_1d(embed_dim_temporal, grid_t)[:, np.newaxis, :]
    pe_t = np.repeat(pe_t, grid_size ** 2, axis=1)
    pe_s = np.repeat(pe_s[np.newaxis, :, :], t_size, axis=0)
    pe = np.concatenate([pe_t, pe_s], axis=-1)
    return pe.reshape([t_size, grid_size, grid_size, embed_dim])


def _tile_of(n, cap=576):
    for t in (cap, 288, 144, 72, 8, 1):
        if n % t == 0 and t <= cap:
            return t
    return 1


def _posadd_body(s_ref, x_ref, p_ref, o_ref):
    o_ref[...] = x_ref[...] + p_ref[...] + s_ref[0, 0]


def _pos_add(x, pos, shift):
    b, p, c = x.shape
    ti = _tile_of(p)
    return pl.pallas_call(
        _posadd_body,
        grid=(b, p // ti),
        in_specs=[
            pl.BlockSpec(memory_space=pltpu.SMEM),
            pl.BlockSpec((1, ti, c), lambda bb, g: (bb, g, 0)),
            pl.BlockSpec((1, ti, c), lambda bb, g: (0, g, 0)),
        ],
        out_specs=pl.BlockSpec((1, ti, c), lambda bb, g: (bb, g, 0)),
        out_shape=jax.ShapeDtypeStruct((b, p, c), x.dtype),
        interpret=_INTERPRET,
    )(shift, x, pos)


def _metric_body(x_ref, a_ref, b_ref):
    blk = x_ref[0]

    def norm_halfmean(base):
        m = blk[:, base:base + _HDIM]
        for hh in range(1, _HEADS):
            m = m + blk[:, base + hh * _HDIM:base + (hh + 1) * _HDIM]
        m = m * (1.0 / _HEADS)
        n = jnp.sqrt(jnp.sum(m * m, axis=1, keepdims=True))
        return m / n

    a_ref[0] = norm_halfmean(0)
    b_ref[0] = norm_halfmean(_MM_HIDDEN)


def _metric(xw3):
    b, t1, c2 = xw3.shape
    ti = _tile_of(t1)
    return pl.pallas_call(
        _metric_body,
        grid=(b, t1 // ti),
        in_specs=[pl.BlockSpec((1, ti, c2), lambda bb, g: (bb, g, 0))],
        out_specs=[
            pl.BlockSpec((1, ti, _HDIM), lambda bb, g: (bb, g, 0)),
            pl.BlockSpec((1, ti, _HDIM), lambda bb, g: (bb, g, 0)),
        ],
        out_shape=[
            jax.ShapeDtypeStruct((b, t1, _HDIM), jnp.float32),
            jax.ShapeDtypeStruct((b, t1, _HDIM), jnp.float32),
        ],
        interpret=_INTERPRET,
    )(xw3)


def _dest_body(a_ref, b_ref, o_ref, *, t1, r):
    A = a_ref[0]
    B = b_ref[0]
    s = lax.dot_general(A, B, (((1,), (1,)), ((), ())),
                        preferred_element_type=jnp.float32)
    mx = jnp.max(s, axis=1, keepdims=True)
    ii = lax.broadcasted_iota(jnp.int32, s.shape, 1)
    idx = jnp.min(jnp.where(s == mx, ii, t1), axis=1, keepdims=True)
    if r >= t1:
        o_ref[0, 0] = idx
    else:
        io = lax.broadcasted_iota(jnp.int32, (t1, t1), 0)
        jj = lax.broadcasted_iota(jnp.int32, (t1, t1), 1)
        diag = jnp.where(io == jj, jnp.broadcast_to(mx, (t1, t1)), 0.0)
        nm_row = lax.dot_general(jnp.ones((1, t1), jnp.float32), diag,
                                 (((1,), (0,)), ((), ())),
                                 preferred_element_type=jnp.float32)
        gt = jnp.sum((nm_row > mx).astype(jnp.int32), axis=1, keepdims=True)
        eq = jnp.sum(((nm_row == mx) & (jj < io)).astype(jnp.int32),
                     axis=1, keepdims=True)
        rank = gt + eq
        o_ref[0, 0] = jnp.where(rank >= r, rank - r, (t1 - r) + idx)


def _dest(mA, mB, r):
    b, t1, _ = mA.shape
    ts = t1 if r < t1 else _tile_of(t1)
    nt = t1 // ts
    out = pl.pallas_call(
        functools.partial(_dest_body, t1=t1, r=r),
        grid=(b, nt),
        in_specs=[
            pl.BlockSpec((1, ts, _HDIM), lambda bb, g: (bb, g, 0)),
            pl.BlockSpec((1, t1, _HDIM), lambda bb, g: (bb, 0, 0)),
        ],
        out_specs=pl.BlockSpec((1, 1, ts, 1), lambda bb, g: (bb, g, 0, 0)),
        out_shape=jax.ShapeDtypeStruct((b, nt, ts, 1), jnp.int32),
        interpret=_INTERPRET,
    )(mA, mB)
    return out.reshape(b, t1)


def _merge_body(dest_ref, xs_ref, xd_ref, se_ref, sd_ref, xo_ref, so_ref,
                *, t1, unm):
    cid = pl.program_id(1)
    if unm:
        xo_ref[0, :unm, :] = jnp.zeros_like(xo_ref[0, :unm, :])
    xo_ref[0, unm:, :] = xd_ref[0]

    def step(i, _):
        d = dest_ref[0, 0, i]
        xo_ref[0, pl.ds(d, 1), :] += xs_ref[0, pl.ds(i, 1), :]
        return 0

    lax.fori_loop(0, t1, step, 0)

    @pl.when(cid == 0)
    def _():
        if unm:
            so_ref[0, :unm, :] = jnp.zeros_like(so_ref[0, :unm, :])
        so_ref[0, unm:, :] = sd_ref[0]

        def step_s(i, _):
            d = dest_ref[0, 0, i]
            so_ref[0, pl.ds(d, 1), :] += se_ref[0, pl.ds(i, 1), :]
            return 0

        lax.fori_loop(0, t1, step_s, 0)


def _merge(xw3, sz3, dest, r):
    b, t1, c2 = xw3.shape
    c = c2 // 2
    unm = t1 - r
    pout = unm + t1
    ck = 256
    nc = c // ck
    xo, so = pl.pallas_call(
        functools.partial(_merge_body, t1=t1, unm=unm),
        grid=(b, nc),
        in_specs=[
            pl.BlockSpec((1, 1, t1), lambda bb, cc: (bb, 0, 0),
                         memory_space=pltpu.SMEM),
            pl.BlockSpec((1, t1, ck), lambda bb, cc: (bb, 0, cc)),
            pl.BlockSpec((1, t1, ck), lambda bb, cc: (bb, 0, nc + cc)),
            pl.BlockSpec((1, t1, 1), lambda bb, cc: (bb, 0, 0)),
            pl.BlockSpec((1, t1, 1), lambda bb, cc: (bb, 0, 0)),
        ],
        out_specs=[
            pl.BlockSpec((1, pout, ck), lambda bb, cc: (bb, 0, cc)),
            pl.BlockSpec((1, pout, 1), lambda bb, cc: (bb, 0, 0)),
        ],
        out_shape=[
            jax.ShapeDtypeStruct((b, pout, c), jnp.float32),
            jax.ShapeDtypeStruct((b, pout, 1), jnp.float32),
        ],
        interpret=_INTERPRET,
    )(dest.reshape(b, 1, t1), xw3, xw3, sz3[:, :, 0:1], sz3[:, :, 1:2])
    return xo, so


def _mlp1_body(x_ref, s_ref, w_ref, b_ref, o_ref):
    xv = x_ref[...] / s_ref[...]
    h = lax.dot_general(xv, w_ref[...], (((1,), (1,)), ((), ())),
                        preferred_element_type=jnp.float32) + b_ref[...]
    o_ref[...] = h * 0.5 * (1.0 + lax.erf(h * np.float32(0.7071067811865476)))


def _mlp2_body(h_ref, w_ref, b_ref, o_ref):
    o_ref[...] = lax.dot_general(h_ref[...], w_ref[...],
                                 (((1,), (1,)), ((), ())),
                                 preferred_element_type=jnp.float32) + b_ref[...]


def _mlp(xw, sz, W1, b1, W2, b2):
    b, p, c = xw.shape
    n = b * p
    x2 = xw.reshape(n, c)
    s2 = sz.reshape(n, 1)
    hdim = W1.shape[0]
    h = pl.pallas_call(
        _mlp1_body,
        grid=(1,),
        in_specs=[
            pl.BlockSpec((n, c), lambda g: (0, 0)),
            pl.BlockSpec((n, 1), lambda g: (0, 0)),
            pl.BlockSpec((hdim, c), lambda g: (0, 0)),
            pl.BlockSpec((1, hdim), lambda g: (0, 0)),
        ],
        out_specs=pl.BlockSpec((n, hdim), lambda g: (0, 0)),
        out_shape=jax.ShapeDtypeStruct((n, hdim), jnp.float32),
        interpret=_INTERPRET,
    )(x2, s2, W1, b1.reshape(1, hdim))
    odim = W2.shape[0]
    tj = 1024
    out = pl.pallas_call(
        _mlp2_body,
        grid=(odim // tj,),
        in_specs=[
            pl.BlockSpec((n, W2.shape[1]), lambda g: (0, 0)),
            pl.BlockSpec((tj, W2.shape[1]), lambda g: (g, 0)),
            pl.BlockSpec((1, tj), lambda g: (0, g)),
        ],
        out_specs=pl.BlockSpec((n, tj), lambda g: (0, g)),
        out_shape=jax.ShapeDtypeStruct((n, odim), jnp.float32),
        interpret=_INTERPRET,
    )(h, W2, b2.reshape(1, odim))
    return out.reshape(b, p, odim)


def kernel(x, local_num_frames, W1, b1, W2, b2):
    b, P, C = x.shape
    h = _HW
    lnf = P // (h * h)
    num_tome = lnf * 16

    pos_np = _pos_3d(_MM_HIDDEN, _HW, _T_MAX).astype(np.float32)
    pos = jnp.asarray(pos_np[:lnf, :h, :h].reshape(1, lnf * h * h, _MM_HIDDEN),
                      x.dtype)
    shift = jnp.asarray(local_num_frames - lnf, x.dtype).reshape(1, 1)

    xw = _pos_add(x, pos, shift)
    sz = jnp.ones((b, P, 1), x.dtype)

    r_list = []
    tmp = P
    while tmp != num_tome:
        if tmp - num_tome <= tmp // 2:
            r_list.append(tmp - num_tome)
            break
        r_list.append(tmp // 2)
        tmp -= tmp // 2

    for r in r_list:
        p = xw.shape[1]
        t1 = p // 2
        r = min(r, t1)
        xw3 = xw.reshape(b, t1, 2 * C)
        sz3 = sz.reshape(b, t1, 2)
        mA, mB = _metric(xw3)
        dest = _dest(mA, mB, r)
        xw, sz = _merge(xw3, sz3, dest, r)

    return _mlp(xw, sz, W1, b1, W2, b2)
